# baseline (speedup 1.0000x reference)
import functools

import jax
import jax.numpy as jnp
from jax import lax
from jax.experimental import pallas as pl
from jax.experimental.pallas import tpu as pltpu
from jax.experimental.pallas import tpu_sc as plsc

E = 8
D = 1024
N = 4096
BM = 128
NP = N + E * BM
NB = NP // BM

_SC_CORES = 2
_SC_SUBCORES = 16
_NW = _SC_CORES * _SC_SUBCORES
_ROWS_PER_W = N // _NW
_CH = 32
_CHUNKS = _ROWS_PER_W // _CH


def _routing_body(a_ref, dest_ref, be_ref, bv_ref):
    a = a_ref[:]
    T = (lax.broadcasted_iota(jnp.int32, (128, 128), 0)
         <= lax.broadcasted_iota(jnp.int32, (128, 128), 1)).astype(jnp.float32)
    m32 = (lax.broadcasted_iota(jnp.int32, (32, 32), 1)
           < lax.broadcasted_iota(jnp.int32, (32, 32), 0)).astype(jnp.float32)
    g = lax.broadcasted_iota(jnp.int32, (1, 128), 1).astype(jnp.float32)

    dest = jnp.zeros((32, 128), jnp.float32)
    be = jnp.zeros((1, 128), jnp.float32)
    bv = jnp.zeros((1, 128), jnp.float32)
    covered = jnp.zeros((1, 128), jnp.float32)
    gs = jnp.float32(0.0)
    for e in range(E):
        ohe = (a == e).astype(jnp.float32)
        incl = jnp.dot(ohe, T, preferred_element_type=jnp.float32)
        s = incl[:, 127:128]
        prev = jnp.dot(m32, s, preferred_element_type=jnp.float32)
        cnt = jnp.sum(ohe)
        rank = incl - ohe + prev
        dest = dest + ohe * (rank + gs)
        aligned = jnp.ceil(cnt / BM) * BM
        start_blk = gs / BM
        end_blk = (gs + aligned) / BM
        in_group = (g >= start_blk) & (g < end_blk)
        has_valid = (g * BM) < (gs + cnt)
        be = be + jnp.where(in_group, jnp.float32(e), 0.0)
        bv = bv + jnp.where(in_group & has_valid, 1.0, 0.0)
        covered = covered + jnp.where(in_group, 1.0, 0.0)
        gs = gs + aligned
    be = be + (1.0 - covered) * jnp.float32(E - 1)
    dest_ref[:] = dest.astype(jnp.int32)
    be_ref[:] = be.astype(jnp.int32)
    bv_ref[:] = bv.astype(jnp.int32)


def _routing(a2):
    return pl.pallas_call(
        _routing_body,
        out_shape=(
            jax.ShapeDtypeStruct((32, 128), jnp.int32),
            jax.ShapeDtypeStruct((1, 128), jnp.int32),
            jax.ShapeDtypeStruct((1, 128), jnp.int32),
        ),
    )(a2)


def _mm_body(be_ref, bv_ref, x_ref, w_ref, b_ref, o_ref, w16_ref):
    i = pl.program_id(0)

    @pl.when((i == 0) | (be_ref[i] != be_ref[jnp.maximum(i - 1, 0)]))
    def _():
        w16_ref[:] = w_ref[0].astype(jnp.bfloat16)

    @pl.when(bv_ref[i] != 0)
    def _():
        x16 = x_ref[:].astype(jnp.bfloat16)
        o_ref[:] = (jnp.dot(x16, w16_ref[:],
                            preferred_element_type=jnp.float32) + b_ref[0])


def _grouped_matmul(be, bv, xs_sorted, W, b3):
    grid_spec = pltpu.PrefetchScalarGridSpec(
        num_scalar_prefetch=2,
        grid=(NB,),
        in_specs=[
            pl.BlockSpec((BM, D), lambda i, be, bv: (i, 0)),
            pl.BlockSpec((1, D, D), lambda i, be, bv: (be[i], 0, 0)),
            pl.BlockSpec((1, 1, D), lambda i, be, bv: (be[i], 0, 0)),
        ],
        out_specs=pl.BlockSpec((BM, D), lambda i, be, bv: (i, 0)),
        scratch_shapes=[pltpu.VMEM((D, D), jnp.bfloat16)],
    )
    return pl.pallas_call(
        _mm_body,
        grid_spec=grid_spec,
        out_shape=jax.ShapeDtypeStruct((NP, D), jnp.float32),
        compiler_params=pltpu.CompilerParams(
            dimension_semantics=("arbitrary",)),
    )(be, bv, xs_sorted, W, b3)


def _sc_mesh():
    return plsc.VectorSubcoreMesh(core_axis_name="c", subcore_axis_name="s",
                                  num_cores=_SC_CORES,
                                  num_subcores=_SC_SUBCORES)


_SC_SCRATCH = [
    pltpu.VMEM((_CHUNKS, _CH), jnp.int32),
    pltpu.VMEM((_CH, D), jnp.float32),
    pltpu.VMEM((_CH, D), jnp.float32),
    pltpu.SemaphoreType.DMA,
    pltpu.SemaphoreType.DMA,
    pltpu.SemaphoreType.DMA,
    pltpu.SemaphoreType.DMA,
]


def _sc_scatter(xs, dest3):
    @functools.partial(
        pl.kernel,
        out_type=jax.ShapeDtypeStruct((NP, D), jnp.float32),
        mesh=_sc_mesh(),
        scratch_types=_SC_SCRATCH,
    )
    def k(xs_hbm, dest_hbm, out_hbm, idx_v, r0, r1, si0, si1, so0, so1):
        wid = lax.axis_index("s") * _SC_CORES + lax.axis_index("c")
        rows = (r0, r1)
        s_in = (si0, si1)
        s_out = (so0, so1)
        pltpu.sync_copy(dest_hbm.at[wid], idx_v)

        def start_in(c):
            base = wid * _ROWS_PER_W + c * _CH
            return pltpu.async_copy(xs_hbm.at[pl.ds(base, _CH), :],
                                    rows[c & 1], s_in[c & 1])

        h_in = {0: start_in(0)}
        h_out = {}
        for c in range(_CHUNKS):
            b = c & 1
            h_in[c].wait()
            if c + 1 < _CHUNKS:
                if c - 1 >= 0:
                    h_out[c - 1].wait()
                h_in[c + 1] = start_in(c + 1)
            h_out[c] = pltpu.async_copy(rows[b], out_hbm.at[idx_v.at[c]],
                                        s_out[b])
        h_out[_CHUNKS - 2].wait()
        h_out[_CHUNKS - 1].wait()

    return k(xs, dest3)


def _sc_gather(ys_sorted, dest3):
    @functools.partial(
        pl.kernel,
        out_type=jax.ShapeDtypeStruct((N, D), jnp.float32),
        mesh=_sc_mesh(),
        scratch_types=_SC_SCRATCH,
    )
    def k(src_hbm, dest_hbm, out_hbm, idx_v, r0, r1, sg0, sg1, so0, so1):
        wid = lax.axis_index("s") * _SC_CORES + lax.axis_index("c")
        rows = (r0, r1)
        s_g = (sg0, sg1)
        s_out = (so0, so1)
        pltpu.sync_copy(dest_hbm.at[wid], idx_v)

        def start_gather(c):
            return pltpu.async_copy(src_hbm.at[idx_v.at[c]], rows[c & 1],
                                    s_g[c & 1])

        h_g = {0: start_gather(0)}
        h_out = {}
        for c in range(_CHUNKS):
            b = c & 1
            h_g[c].wait()
            if c + 1 < _CHUNKS:
                if c - 1 >= 0:
                    h_out[c - 1].wait()
                h_g[c + 1] = start_gather(c + 1)
            base = wid * _ROWS_PER_W + c * _CH
            h_out[c] = pltpu.async_copy(rows[b],
                                        out_hbm.at[pl.ds(base, _CH), :],
                                        s_out[b])
        h_out[_CHUNKS - 2].wait()
        h_out[_CHUNKS - 1].wait()

    return k(ys_sorted, dest3)


def kernel(xs, mxs, actions, W, b):
    a2 = actions.astype(jnp.int32).reshape(32, 128)
    dest2, be2, bv2 = _routing(a2)
    dest3 = dest2.reshape(_NW, _CHUNKS, _CH)
    be = be2.reshape(128)[:NB]
    bv = bv2.reshape(128)[:NB]
    xs_sorted = _sc_scatter(xs, dest3)
    ys_sorted = _grouped_matmul(be, bv, xs_sorted, W, b.reshape(E, 1, D))
    ys = _sc_gather(ys_sorted, dest3)
    return (ys, mxs, actions)


# speedup vs baseline: 1.0690x; 1.0690x over previous
import functools

import jax
import jax.numpy as jnp
from jax import lax
from jax.experimental import pallas as pl
from jax.experimental.pallas import tpu as pltpu
from jax.experimental.pallas import tpu_sc as plsc

E = 8
D = 1024
N = 4096
BM = 256
NP = N + E * BM
NB = NP // BM

_SC_CORES = 2
_SC_SUBCORES = 16
_NW = _SC_CORES * _SC_SUBCORES
_ROWS_PER_W = N // _NW
_CH = 32
_CHUNKS = _ROWS_PER_W // _CH


def _routing_body(a_ref, dest_ref, be_ref, bv_ref):
    a = a_ref[:]
    T = (lax.broadcasted_iota(jnp.int32, (128, 128), 0)
         <= lax.broadcasted_iota(jnp.int32, (128, 128), 1)).astype(jnp.float32)
    m32 = (lax.broadcasted_iota(jnp.int32, (32, 32), 1)
           < lax.broadcasted_iota(jnp.int32, (32, 32), 0)).astype(jnp.float32)
    g = lax.broadcasted_iota(jnp.int32, (1, 128), 1).astype(jnp.float32)

    dest = jnp.zeros((32, 128), jnp.float32)
    be = jnp.zeros((1, 128), jnp.float32)
    bv = jnp.zeros((1, 128), jnp.float32)
    covered = jnp.zeros((1, 128), jnp.float32)
    gs = jnp.float32(0.0)
    for e in range(E):
        ohe = (a == e).astype(jnp.float32)
        incl = jnp.dot(ohe, T, preferred_element_type=jnp.float32)
        s = incl[:, 127:128]
        prev = jnp.dot(m32, s, preferred_element_type=jnp.float32)
        cnt = jnp.sum(ohe)
        rank = incl - ohe + prev
        dest = dest + ohe * (rank + gs)
        aligned = jnp.ceil(cnt / BM) * BM
        start_blk = gs / BM
        end_blk = (gs + aligned) / BM
        in_group = (g >= start_blk) & (g < end_blk)
        has_valid = (g * BM) < (gs + cnt)
        be = be + jnp.where(in_group, jnp.float32(e), 0.0)
        bv = bv + jnp.where(in_group & has_valid, 1.0, 0.0)
        covered = covered + jnp.where(in_group, 1.0, 0.0)
        gs = gs + aligned
    be = be + (1.0 - covered) * jnp.float32(E - 1)
    dest_ref[:] = dest.astype(jnp.int32)
    be_ref[:] = be.astype(jnp.int32)
    bv_ref[:] = bv.astype(jnp.int32)


def _routing(a2):
    return pl.pallas_call(
        _routing_body,
        out_shape=(
            jax.ShapeDtypeStruct((32, 128), jnp.int32),
            jax.ShapeDtypeStruct((1, 128), jnp.int32),
            jax.ShapeDtypeStruct((1, 128), jnp.int32),
        ),
    )(a2)


def _mm_body(be_ref, bv_ref, x_ref, w_ref, b_ref, o_ref, w16_ref):
    i = pl.program_id(0)

    @pl.when((i == 0) | (be_ref[i] != be_ref[jnp.maximum(i - 1, 0)]))
    def _():
        w16_ref[:] = w_ref[0].astype(jnp.bfloat16)

    @pl.when(bv_ref[i] != 0)
    def _():
        x16 = x_ref[:].astype(jnp.bfloat16)
        o_ref[:] = (jnp.dot(x16, w16_ref[:],
                            preferred_element_type=jnp.float32) + b_ref[0])


def _grouped_matmul(be, bv, xs_sorted, W, b3):
    grid_spec = pltpu.PrefetchScalarGridSpec(
        num_scalar_prefetch=2,
        grid=(NB,),
        in_specs=[
            pl.BlockSpec((BM, D), lambda i, be, bv: (i, 0)),
            pl.BlockSpec((1, D, D), lambda i, be, bv: (be[i], 0, 0)),
            pl.BlockSpec((1, 1, D), lambda i, be, bv: (be[i], 0, 0)),
        ],
        out_specs=pl.BlockSpec((BM, D), lambda i, be, bv: (i, 0)),
        scratch_shapes=[pltpu.VMEM((D, D), jnp.bfloat16)],
    )
    return pl.pallas_call(
        _mm_body,
        grid_spec=grid_spec,
        out_shape=jax.ShapeDtypeStruct((NP, D), jnp.float32),
        compiler_params=pltpu.CompilerParams(
            dimension_semantics=("arbitrary",)),
    )(be, bv, xs_sorted, W, b3)


def _sc_mesh():
    return plsc.VectorSubcoreMesh(core_axis_name="c", subcore_axis_name="s",
                                  num_cores=_SC_CORES,
                                  num_subcores=_SC_SUBCORES)


_SC_SCRATCH = [
    pltpu.VMEM((_CHUNKS, _CH), jnp.int32),
    pltpu.VMEM((_CH, D), jnp.float32),
    pltpu.VMEM((_CH, D), jnp.float32),
    pltpu.SemaphoreType.DMA,
    pltpu.SemaphoreType.DMA,
    pltpu.SemaphoreType.DMA,
    pltpu.SemaphoreType.DMA,
]


def _sc_scatter(xs, dest3):
    @functools.partial(
        pl.kernel,
        out_type=jax.ShapeDtypeStruct((NP, D), jnp.float32),
        mesh=_sc_mesh(),
        scratch_types=_SC_SCRATCH,
    )
    def k(xs_hbm, dest_hbm, out_hbm, idx_v, r0, r1, si0, si1, so0, so1):
        wid = lax.axis_index("s") * _SC_CORES + lax.axis_index("c")
        rows = (r0, r1)
        s_in = (si0, si1)
        s_out = (so0, so1)
        pltpu.sync_copy(dest_hbm.at[wid], idx_v)

        def start_in(c):
            base = wid * _ROWS_PER_W + c * _CH
            return pltpu.async_copy(xs_hbm.at[pl.ds(base, _CH), :],
                                    rows[c & 1], s_in[c & 1])

        h_in = {0: start_in(0)}
        h_out = {}
        for c in range(_CHUNKS):
            b = c & 1
            h_in[c].wait()
            if c + 1 < _CHUNKS:
                if c - 1 >= 0:
                    h_out[c - 1].wait()
                h_in[c + 1] = start_in(c + 1)
            h_out[c] = pltpu.async_copy(rows[b], out_hbm.at[idx_v.at[c]],
                                        s_out[b])
        h_out[_CHUNKS - 2].wait()
        h_out[_CHUNKS - 1].wait()

    return k(xs, dest3)


def _sc_gather(ys_sorted, dest3):
    @functools.partial(
        pl.kernel,
        out_type=jax.ShapeDtypeStruct((N, D), jnp.float32),
        mesh=_sc_mesh(),
        scratch_types=_SC_SCRATCH,
    )
    def k(src_hbm, dest_hbm, out_hbm, idx_v, r0, r1, sg0, sg1, so0, so1):
        wid = lax.axis_index("s") * _SC_CORES + lax.axis_index("c")
        rows = (r0, r1)
        s_g = (sg0, sg1)
        s_out = (so0, so1)
        pltpu.sync_copy(dest_hbm.at[wid], idx_v)

        def start_gather(c):
            return pltpu.async_copy(src_hbm.at[idx_v.at[c]], rows[c & 1],
                                    s_g[c & 1])

        h_g = {0: start_gather(0)}
        h_out = {}
        for c in range(_CHUNKS):
            b = c & 1
            h_g[c].wait()
            if c + 1 < _CHUNKS:
                if c - 1 >= 0:
                    h_out[c - 1].wait()
                h_g[c + 1] = start_gather(c + 1)
            base = wid * _ROWS_PER_W + c * _CH
            h_out[c] = pltpu.async_copy(rows[b],
                                        out_hbm.at[pl.ds(base, _CH), :],
                                        s_out[b])
        h_out[_CHUNKS - 2].wait()
        h_out[_CHUNKS - 1].wait()

    return k(ys_sorted, dest3)


def kernel(xs, mxs, actions, W, b):
    a2 = actions.astype(jnp.int32).reshape(32, 128)
    dest2, be2, bv2 = _routing(a2)
    dest3 = dest2.reshape(_NW, _CHUNKS, _CH)
    be = be2.reshape(128)[:NB]
    bv = bv2.reshape(128)[:NB]
    xs_sorted = _sc_scatter(xs, dest3)
    ys_sorted = _grouped_matmul(be, bv, xs_sorted, W, b.reshape(E, 1, D))
    ys = _sc_gather(ys_sorted, dest3)
    return (ys, mxs, actions)


# speedup vs baseline: 1.1732x; 1.0975x over previous
import functools

import jax
import jax.numpy as jnp
from jax import lax
from jax.experimental import pallas as pl
from jax.experimental.pallas import tpu as pltpu
from jax.experimental.pallas import tpu_sc as plsc

E = 8
D = 1024
N = 4096
BM = 256
NP = N + E * BM
NB = NP // BM

_SC_CORES = 2
_SC_SUBCORES = 16
_NW = _SC_CORES * _SC_SUBCORES
_ROWS_PER_W = N // _NW
_CH = 64
_CHUNKS = _ROWS_PER_W // _CH


def _routing_body(a_ref, dest_ref, be_ref, bv_ref, src_ref):
    a = a_ref[:]
    T = (lax.broadcasted_iota(jnp.int32, (128, 128), 0)
         <= lax.broadcasted_iota(jnp.int32, (128, 128), 1)).astype(jnp.float32)
    m32 = (lax.broadcasted_iota(jnp.int32, (32, 32), 1)
           < lax.broadcasted_iota(jnp.int32, (32, 32), 0)).astype(jnp.float32)
    g = lax.broadcasted_iota(jnp.int32, (1, 128), 1).astype(jnp.float32)

    dest = jnp.zeros((32, 128), jnp.float32)
    be = jnp.zeros((1, 128), jnp.float32)
    bv = jnp.zeros((1, 128), jnp.float32)
    src = jnp.zeros((1, 128), jnp.float32)
    covered = jnp.zeros((1, 128), jnp.float32)
    gs = jnp.float32(0.0)
    last_valid = jnp.float32(0.0)
    for e in range(E):
        ohe = (a == e).astype(jnp.float32)
        incl = jnp.dot(ohe, T, preferred_element_type=jnp.float32)
        s = incl[:, 127:128]
        prev = jnp.dot(m32, s, preferred_element_type=jnp.float32)
        cnt = jnp.sum(ohe)
        rank = incl - ohe + prev
        dest = dest + ohe * (rank + gs)
        aligned = jnp.ceil(cnt / BM) * BM
        vblk = aligned / BM
        start_blk = gs / BM
        end_blk = start_blk + vblk
        in_group = (g >= start_blk) & (g < end_blk)
        has_valid = (g * BM) < (gs + cnt)
        be = be + jnp.where(in_group, jnp.float32(e), 0.0)
        bv = bv + jnp.where(in_group & has_valid, 1.0, 0.0)
        grp_last = jnp.maximum(start_blk + jnp.ceil(cnt / BM) - 1.0, 0.0)
        src = src + jnp.where(in_group,
                              jnp.where(has_valid, g, grp_last), 0.0)
        covered = covered + jnp.where(in_group, 1.0, 0.0)
        last_valid = jnp.where(cnt > 0, grp_last, last_valid)
        gs = gs + aligned
    be = be + (1.0 - covered) * jnp.float32(E - 1)
    src = src + (1.0 - covered) * last_valid
    dest_ref[:] = dest.astype(jnp.int32)
    be_ref[:] = be.astype(jnp.int32)
    bv_ref[:] = bv.astype(jnp.int32)
    src_ref[:] = src.astype(jnp.int32)


def _routing(a2):
    return pl.pallas_call(
        _routing_body,
        out_shape=(
            jax.ShapeDtypeStruct((32, 128), jnp.int32),
            jax.ShapeDtypeStruct((1, 128), jnp.int32),
            jax.ShapeDtypeStruct((1, 128), jnp.int32),
            jax.ShapeDtypeStruct((1, 128), jnp.int32),
        ),
    )(a2)


def _mm_body(be_ref, bv_ref, src_ref, x_ref, w_ref, b_ref, o_ref):
    i = pl.program_id(0)

    @pl.when(bv_ref[i] != 0)
    def _():
        o_ref[:] = (jnp.dot(x_ref[:], w_ref[0],
                            preferred_element_type=jnp.float32) + b_ref[0])


def _grouped_matmul(be, bv, src, xs_sorted, W, b3):
    grid_spec = pltpu.PrefetchScalarGridSpec(
        num_scalar_prefetch=3,
        grid=(NB,),
        in_specs=[
            pl.BlockSpec((BM, D), lambda i, be, bv, src: (src[i], 0)),
            pl.BlockSpec((1, D, D), lambda i, be, bv, src: (be[i], 0, 0)),
            pl.BlockSpec((1, 1, D), lambda i, be, bv, src: (be[i], 0, 0)),
        ],
        out_specs=pl.BlockSpec((BM, D), lambda i, be, bv, src: (src[i], 0)),
    )
    return pl.pallas_call(
        _mm_body,
        grid_spec=grid_spec,
        out_shape=jax.ShapeDtypeStruct((NP, D), jnp.float32),
        compiler_params=pltpu.CompilerParams(
            dimension_semantics=("arbitrary",)),
    )(be, bv, src, xs_sorted, W, b3)


def _sc_mesh():
    return plsc.VectorSubcoreMesh(core_axis_name="c", subcore_axis_name="s",
                                  num_cores=_SC_CORES,
                                  num_subcores=_SC_SUBCORES)


_SC_SCRATCH = [
    pltpu.VMEM((_CH,), jnp.int32),
    pltpu.VMEM((_CH, D), jnp.float32),
    pltpu.SemaphoreType.DMA,
]


def _sc_scatter(xs, dest):
    @functools.partial(
        pl.kernel,
        out_type=jax.ShapeDtypeStruct((NP, D), jnp.float32),
        mesh=_sc_mesh(),
        scratch_types=_SC_SCRATCH,
    )
    def k(xs_hbm, dest_hbm, out_hbm, idx_v, rows_v, sem):
        wid = lax.axis_index("s") * _SC_CORES + lax.axis_index("c")
        for c in range(_CHUNKS):
            base = wid * _ROWS_PER_W + c * _CH
            pltpu.sync_copy(dest_hbm.at[pl.ds(base, _CH)], idx_v)
            pltpu.sync_copy(xs_hbm.at[pl.ds(base, _CH), :], rows_v)
            pltpu.async_copy(rows_v, out_hbm.at[idx_v], sem).wait()

    return k(xs, dest)


def _sc_gather(ys_sorted, dest):
    @functools.partial(
        pl.kernel,
        out_type=jax.ShapeDtypeStruct((N, D), jnp.float32),
        mesh=_sc_mesh(),
        scratch_types=_SC_SCRATCH,
    )
    def k(src_hbm, dest_hbm, out_hbm, idx_v, rows_v, sem):
        wid = lax.axis_index("s") * _SC_CORES + lax.axis_index("c")
        for c in range(_CHUNKS):
            base = wid * _ROWS_PER_W + c * _CH
            pltpu.sync_copy(dest_hbm.at[pl.ds(base, _CH)], idx_v)
            pltpu.async_copy(src_hbm.at[idx_v], rows_v, sem).wait()
            pltpu.sync_copy(rows_v, out_hbm.at[pl.ds(base, _CH), :])

    return k(ys_sorted, dest)


def kernel(xs, mxs, actions, W, b):
    a2 = actions.astype(jnp.int32).reshape(32, 128)
    dest2, be2, bv2, src2 = _routing(a2)
    dest = dest2.reshape(N)
    be = be2.reshape(128)[:NB]
    bv = bv2.reshape(128)[:NB]
    src = src2.reshape(128)[:NB]
    xs_sorted = _sc_scatter(xs, dest)
    ys_sorted = _grouped_matmul(be, bv, src, xs_sorted, W, b.reshape(E, 1, D))
    ys = _sc_gather(ys_sorted, dest)
    return (ys, mxs, actions)


# speedup vs baseline: 1.1936x; 1.0174x over previous
import functools

import jax
import jax.numpy as jnp
from jax import lax
from jax.experimental import pallas as pl
from jax.experimental.pallas import tpu as pltpu
from jax.experimental.pallas import tpu_sc as plsc

E = 8
D = 1024
N = 4096
BM = 256
NP = N + E * BM
NB = NP // BM

_SC_CORES = 2
_SC_SUBCORES = 16
_NW = _SC_CORES * _SC_SUBCORES
_ROWS_PER_W = N // _NW
_CH = 64
_CHUNKS = _ROWS_PER_W // _CH


def _routing_body(a_ref, dest_ref, be_ref, bv_ref, src_ref):
    a = a_ref[:]
    T = (lax.broadcasted_iota(jnp.int32, (128, 128), 0)
         <= lax.broadcasted_iota(jnp.int32, (128, 128), 1)).astype(jnp.float32)
    m32 = (lax.broadcasted_iota(jnp.int32, (32, 32), 1)
           < lax.broadcasted_iota(jnp.int32, (32, 32), 0)).astype(jnp.float32)
    g = lax.broadcasted_iota(jnp.int32, (1, 128), 1).astype(jnp.float32)

    dest = jnp.zeros((32, 128), jnp.float32)
    be = jnp.zeros((1, 128), jnp.float32)
    bv = jnp.zeros((1, 128), jnp.float32)
    src = jnp.zeros((1, 128), jnp.float32)
    covered = jnp.zeros((1, 128), jnp.float32)
    gs = jnp.float32(0.0)
    last_valid = jnp.float32(0.0)
    for e in range(E):
        ohe = (a == e).astype(jnp.float32)
        incl = jnp.dot(ohe, T, preferred_element_type=jnp.float32)
        s = incl[:, 127:128]
        prev = jnp.dot(m32, s, preferred_element_type=jnp.float32)
        cnt = jnp.sum(ohe)
        rank = incl - ohe + prev
        dest = dest + ohe * (rank + gs)
        aligned = jnp.ceil(cnt / BM) * BM
        vblk = aligned / BM
        start_blk = gs / BM
        end_blk = start_blk + vblk
        in_group = (g >= start_blk) & (g < end_blk)
        has_valid = (g * BM) < (gs + cnt)
        be = be + jnp.where(in_group, jnp.float32(e), 0.0)
        bv = bv + jnp.where(in_group & has_valid, 1.0, 0.0)
        grp_last = jnp.maximum(start_blk + jnp.ceil(cnt / BM) - 1.0, 0.0)
        src = src + jnp.where(in_group,
                              jnp.where(has_valid, g, grp_last), 0.0)
        covered = covered + jnp.where(in_group, 1.0, 0.0)
        last_valid = jnp.where(cnt > 0, grp_last, last_valid)
        gs = gs + aligned
    be = be + (1.0 - covered) * jnp.float32(E - 1)
    src = src + (1.0 - covered) * last_valid
    dest_ref[:] = dest.astype(jnp.int32)
    be_ref[:] = be.astype(jnp.int32)
    bv_ref[:] = bv.astype(jnp.int32)
    src_ref[:] = src.astype(jnp.int32)


def _routing(a2):
    return pl.pallas_call(
        _routing_body,
        out_shape=(
            jax.ShapeDtypeStruct((32, 128), jnp.int32),
            jax.ShapeDtypeStruct((1, 128), jnp.int32),
            jax.ShapeDtypeStruct((1, 128), jnp.int32),
            jax.ShapeDtypeStruct((1, 128), jnp.int32),
        ),
    )(a2)


def _mm_body(be_ref, bv_ref, src_ref, x_ref, w_ref, b_ref, o_ref):
    i = pl.program_id(0)

    @pl.when(bv_ref[i] != 0)
    def _():
        e = be_ref[i]
        o_ref[:] = (jnp.dot(x_ref[:], w_ref[e],
                            preferred_element_type=jnp.float32) + b_ref[e])


def _grouped_matmul(be, bv, src, xs_sorted, W, b3):
    grid_spec = pltpu.PrefetchScalarGridSpec(
        num_scalar_prefetch=3,
        grid=(NB,),
        in_specs=[
            pl.BlockSpec((BM, D), lambda i, be, bv, src: (src[i], 0)),
            pl.BlockSpec((E, D, D), lambda i, be, bv, src: (0, 0, 0)),
            pl.BlockSpec((E, 1, D), lambda i, be, bv, src: (0, 0, 0)),
        ],
        out_specs=pl.BlockSpec((BM, D), lambda i, be, bv, src: (src[i], 0)),
    )
    return pl.pallas_call(
        _mm_body,
        grid_spec=grid_spec,
        out_shape=jax.ShapeDtypeStruct((NP, D), jnp.float32),
        compiler_params=pltpu.CompilerParams(
            dimension_semantics=("arbitrary",)),
    )(be, bv, src, xs_sorted, W, b3)


def _sc_mesh():
    return plsc.VectorSubcoreMesh(core_axis_name="c", subcore_axis_name="s",
                                  num_cores=_SC_CORES,
                                  num_subcores=_SC_SUBCORES)


_SC_SCRATCH = [
    pltpu.VMEM((_CH,), jnp.int32),
    pltpu.VMEM((_CH, D), jnp.float32),
    pltpu.SemaphoreType.DMA,
]


def _sc_scatter(xs, dest):
    @functools.partial(
        pl.kernel,
        out_type=jax.ShapeDtypeStruct((NP, D), jnp.float32),
        mesh=_sc_mesh(),
        scratch_types=_SC_SCRATCH,
    )
    def k(xs_hbm, dest_hbm, out_hbm, idx_v, rows_v, sem):
        wid = lax.axis_index("s") * _SC_CORES + lax.axis_index("c")
        for c in range(_CHUNKS):
            base = wid * _ROWS_PER_W + c * _CH
            pltpu.sync_copy(dest_hbm.at[pl.ds(base, _CH)], idx_v)
            pltpu.sync_copy(xs_hbm.at[pl.ds(base, _CH), :], rows_v)
            pltpu.async_copy(rows_v, out_hbm.at[idx_v], sem).wait()

    return k(xs, dest)


def _sc_gather(ys_sorted, dest):
    @functools.partial(
        pl.kernel,
        out_type=jax.ShapeDtypeStruct((N, D), jnp.float32),
        mesh=_sc_mesh(),
        scratch_types=_SC_SCRATCH,
    )
    def k(src_hbm, dest_hbm, out_hbm, idx_v, rows_v, sem):
        wid = lax.axis_index("s") * _SC_CORES + lax.axis_index("c")
        for c in range(_CHUNKS):
            base = wid * _ROWS_PER_W + c * _CH
            pltpu.sync_copy(dest_hbm.at[pl.ds(base, _CH)], idx_v)
            pltpu.async_copy(src_hbm.at[idx_v], rows_v, sem).wait()
            pltpu.sync_copy(rows_v, out_hbm.at[pl.ds(base, _CH), :])

    return k(ys_sorted, dest)


def kernel(xs, mxs, actions, W, b):
    a2 = actions.astype(jnp.int32).reshape(32, 128)
    dest2, be2, bv2, src2 = _routing(a2)
    dest = dest2.reshape(N)
    be = be2.reshape(128)[:NB]
    bv = bv2.reshape(128)[:NB]
    src = src2.reshape(128)[:NB]
    xs_sorted = _sc_scatter(xs, dest)
    ys_sorted = _grouped_matmul(be, bv, src, xs_sorted, W, b.reshape(E, 1, D))
    ys = _sc_gather(ys_sorted, dest)
    return (ys, mxs, actions)
